# Initial kernel scaffold; baseline (speedup 1.0000x reference)
import jax
import jax.numpy as jnp
from jax.experimental import pallas as pl


def kernel(bev_feats, w1, bn_gamma, bn_beta, w2, b2, w_ce, b_ce, bev_pos):
    raise NotImplementedError("write your pallas kernel here")


# speedup vs baseline: 1.9697x; 1.9697x over previous
import functools

import jax
import jax.numpy as jnp
from jax import lax
from jax.experimental import pallas as pl
from jax.experimental.pallas import tpu as pltpu
from jax.experimental.pallas import tpu_sc as plsc

B = 2
C_IN = 128
H = 256
W = 256
NUM_CLASSES = 10
NUM_PROPOSALS = 500
HW = H * W
N_SEL = NUM_CLASSES * HW

R = 8
NB = H // R

NS = 16
CHUNK = N_SEL // NS
NBINS = 2048
CAP = 1024
CAND = 1088
NP_PAD = 512
HITCAP = 1024


def _conv_nms_body(x_prev, x_cur, x_next, w1r, w2r, scale, beta, b2,
                   dense_out, hm_out, halo, y1):
    i = pl.program_id(1)
    f32 = jnp.float32
    bf16 = jnp.bfloat16

    zcol = jnp.zeros((R + 6, 1, C_IN), f32)
    halo[:, 0:1, :] = zcol
    halo[:, 257:258, :] = zcol
    halo[3:3 + R, 1:257, :] = x_cur[0]
    halo[0:3, 1:257, :] = jnp.where(i > 0, x_prev[0, R - 3:R, :, :], 0.0)
    halo[3 + R:R + 6, 1:257, :] = jnp.where(i < NB - 1, x_next[0, 0:3, :, :], 0.0)

    a9 = jnp.concatenate(
        [halo[t // 3:t // 3 + R + 4, t % 3:t % 3 + 256, :] for t in range(9)],
        axis=2).astype(bf16)
    acc = lax.dot_general(a9, w1r[:].astype(bf16), (((2,), (0,)), ((), ())),
                          preferred_element_type=f32)
    y1c = jnp.maximum(acc * scale[:][None, None, :] + beta[:][None, None, :], 0.0)
    gr = i * R - 2 + lax.broadcasted_iota(jnp.int32, (R + 4, 1, 1), 0)
    y1c = jnp.where((gr >= 0) & (gr < H), y1c, 0.0)
    zcol1 = jnp.zeros((R + 4, 1, C_IN), f32)
    y1[:, 0:1, :] = zcol1
    y1[:, 257:258, :] = zcol1
    y1[:, 1:257, :] = y1c

    a9b = jnp.concatenate(
        [y1[t // 3:t // 3 + R + 2, t % 3:t % 3 + 256, :] for t in range(9)],
        axis=2).astype(bf16)
    acc2 = lax.dot_general(a9b, w2r[:].astype(bf16), (((2,), (0,)), ((), ())),
                           preferred_element_type=f32)
    y2 = acc2 + b2[:][None, None, :]
    dense_out[0] = y2[1:R + 1]

    s = jax.nn.sigmoid(y2)
    sp = jnp.pad(s, ((0, 0), (1, 1), (0, 0)))
    m = None
    for t in range(9):
        v = sp[t // 3:t // 3 + R, t % 3:t % 3 + 256, :]
        m = v if m is None else jnp.maximum(m, v)
    hmc = s[1:R + 1]
    grow = i * R + lax.broadcasted_iota(jnp.int32, (R, 1, 1), 0)
    interior_r = (grow >= 1) & (grow <= H - 2)
    col = lax.broadcasted_iota(jnp.int32, (1, 256, 1), 1)
    interior_c = (col >= 1) & (col <= W - 2)
    cls = lax.broadcasted_iota(jnp.int32, (1, 1, NUM_CLASSES), 2)
    keep = interior_r & interior_c & (hmc >= m)
    masked = jnp.where(keep, hmc, 0.0)
    hm_out[0] = jnp.where(cls >= 8, hmc, masked)


def _conv_nms(x_nhwc, w1r, w2r, scale, beta, b2):
    f32 = jnp.float32
    blk = lambda b, i: (b, i, 0, 0)
    return pl.pallas_call(
        _conv_nms_body,
        grid=(B, NB),
        in_specs=[
            pl.BlockSpec((1, R, W, C_IN), lambda b, i: (b, jnp.maximum(i - 1, 0), 0, 0)),
            pl.BlockSpec((1, R, W, C_IN), blk),
            pl.BlockSpec((1, R, W, C_IN), lambda b, i: (b, jnp.minimum(i + 1, NB - 1), 0, 0)),
            pl.BlockSpec((9 * C_IN, C_IN), lambda b, i: (0, 0)),
            pl.BlockSpec((9 * C_IN, NUM_CLASSES), lambda b, i: (0, 0)),
            pl.BlockSpec((C_IN,), lambda b, i: (0,)),
            pl.BlockSpec((C_IN,), lambda b, i: (0,)),
            pl.BlockSpec((NUM_CLASSES,), lambda b, i: (0,)),
        ],
        out_specs=[
            pl.BlockSpec((1, R, W, NUM_CLASSES), blk),
            pl.BlockSpec((1, R, W, NUM_CLASSES), blk),
        ],
        out_shape=[
            jax.ShapeDtypeStruct((B, H, W, NUM_CLASSES), f32),
            jax.ShapeDtypeStruct((B, H, W, NUM_CLASSES), f32),
        ],
        scratch_shapes=[
            pltpu.VMEM((R + 6, 258, C_IN), f32),
            pltpu.VMEM((R + 4, 258, C_IN), f32),
        ],
    )(x_nhwc, x_nhwc, x_nhwc, w1r, w2r, scale, beta, b2)


def _thresh_body(x_ref, out_ref):
    f32 = jnp.float32
    x = x_ref[0]

    def body(k, c):
        lo, hi = c
        t = (lo + hi) * 0.5
        cnt = jnp.sum((x >= t).astype(f32))
        big = cnt >= float(NUM_PROPOSALS)
        return (jnp.where(big, t, lo), jnp.where(big, hi, t))

    lo, _ = lax.fori_loop(0, 35, body, (jnp.float32(0.0), jnp.float32(1.0)))
    out_ref[0, 0] = jnp.full((128,), lo, f32)


def _thresholds(hm_sel3):
    return pl.pallas_call(
        _thresh_body,
        grid=(B,),
        in_specs=[pl.BlockSpec((1, 640, 1024), lambda b: (b, 0, 0))],
        out_specs=pl.BlockSpec((1, 1, 128), lambda b: (b, 0, 0)),
        out_shape=jax.ShapeDtypeStruct((B, 1, 128), jnp.float32),
    )(hm_sel3)


def _sc_body(hm_sel, thr, xb, wceT, bce,
             qf, qpos, qlab, qsp,
             v_chunk,
             cand_v, cand_i, sorted_l,
             st_a, st_b, st_c, st_d, hitlist,
             rows_xb, rows_wce, bce_l,
             sp_cval, sp_cidx, sp_sorted,
             cnt_smem):
    i32 = jnp.int32
    f32 = jnp.float32
    b = lax.axis_index("c")
    s = lax.axis_index("s")
    lane = lax.iota(i32, 16)

    pltpu.sync_copy(bce, bce_l)
    pltpu.sync_copy(hm_sel.at[b, pl.ds(s * CHUNK, CHUNK)], v_chunk)

    pltpu.sync_copy(thr.at[b, pl.ds(0, 16)], st_d)
    thr_v = st_d[...]

    @pl.when(s == 0)
    def _():
        st_b[...] = jnp.full((16,), 1 << 28, i32)
        st_c[...] = jnp.zeros((16,), i32)
        st_d[...] = jnp.full((16,), -1.0, f32)
        for k in range(CAND // 16):
            pltpu.sync_copy(st_b, sp_cidx.at[pl.ds(k * 16, 16)])
            pltpu.sync_copy(st_d, sp_cval.at[pl.ds(k * 16, 16)])
        for k in range(NP_PAD // 16):
            pltpu.sync_copy(st_c, sp_sorted.at[pl.ds(k * 16, 16)])
        cnt_smem[0] = 0
    plsc.subcore_barrier()

    def _vsum(vec_i32):
        t = vec_i32[0]
        for l in range(1, 16):
            t = t + vec_i32[l]
        return t

    def c_body(j, carry):
        acc, nhit = carry
        v = v_chunk[pl.ds(j * 16, 16)]
        m01 = jnp.where(v >= thr_v, 1, 0)
        c16 = _vsum(m01)

        @pl.when(c16 > 0)
        def _():
            hitlist[pl.ds(jnp.minimum(nhit, HITCAP - 1) * 16, 16)] = (
                jnp.full((16,), j, i32))
        nhit = jnp.minimum(nhit + jnp.where(c16 > 0, 1, 0), HITCAP - 1)
        return (acc + m01, nhit)
    cnt_vec, nhit = lax.fori_loop(0, CHUNK // 16, c_body,
                                  (jnp.zeros((16,), i32), 0))
    cnt = _vsum(cnt_vec)
    my_off = plsc.fetch_and_add(cnt_smem.at[0], cnt, subcore_id=0)

    def k_body(h, cur):
        j = hitlist[pl.ds(h * 16, 16)][0]
        v = v_chunk[pl.ds(j * 16, 16)]
        mask = v >= thr_v
        m01 = jnp.where(mask, 1, 0)
        gidx = s * CHUNK + j * 16 + lane
        pf = jnp.zeros((16,), i32)
        run = jnp.int32(0)
        for l in range(16):
            pf = jnp.where(lane == l, run, pf)
            run = run + m01[l]
        pos = my_off + cur + pf
        tgt = jnp.where(mask & (pos < CAP), pos, CAP + (pos & 63))
        st_a[...] = tgt
        st_d[...] = v
        pltpu.sync_copy(st_d, sp_cval.at[st_a])
        st_b[...] = gidx
        pltpu.sync_copy(st_b, sp_cidx.at[st_a])
        return cur + run
    lax.fori_loop(0, nhit, k_body, 0)
    plsc.subcore_barrier()

    pltpu.sync_copy(sp_cval, cand_v)
    pltpu.sync_copy(sp_cidx, cand_i)
    p0 = s * 64

    def pv_body(pv, _):
        vch = cand_v[pl.ds(p0 + pv * 16, 16)]
        ich = cand_i[pl.ds(p0 + pv * 16, 16)]
        rank_vec = jnp.zeros((16,), i32)
        for pj in range(16):
            vp = vch[pj]
            ip = ich[pj]

            def q_body(q, acc, vp=vp, ip=ip):
                vq = cand_v[pl.ds(q * 16, 16)]
                iq = cand_i[pl.ds(q * 16, 16)]
                pri = (vq > vp) | ((vq == vp) & (iq < ip))
                return acc + jnp.where(pri, 1, 0)
            acc = lax.fori_loop(0, CAND // 16, q_body, jnp.zeros((16,), i32))
            rank = _vsum(acc)
            rank_vec = jnp.where(lane == pj, rank, rank_vec)

        tgt = jnp.where(rank_vec < NUM_PROPOSALS, rank_vec,
                        NUM_PROPOSALS + (lane & 7))
        st_a[...] = tgt
        st_b[...] = ich
        pltpu.sync_copy(st_b, sp_sorted.at[st_a])
        return 0
    lax.fori_loop(0, 4, pv_body, 0)
    plsc.subcore_barrier()

    pltpu.sync_copy(sp_sorted, sorted_l)
    for ch in range(2):
        base = (s * 2 + ch) * 16
        prop = sorted_l[pl.ds(base, 16)]
        sp_i = prop & (HW - 1)
        labv = lax.shift_right_logical(prop, 16)

        st_b[...] = labv
        pltpu.sync_copy(st_b, qlab.at[b, pl.ds(base, 16)])

        xv = (lax.shift_right_logical(sp_i, 8)).astype(f32) + 0.5
        yv = (sp_i & (W - 1)).astype(f32) + 0.5
        st_d[...] = xv
        pltpu.sync_copy(st_d, qpos.at[b, 0, pl.ds(base, 16)])
        st_d[...] = yv
        pltpu.sync_copy(st_d, qpos.at[b, 1, pl.ds(base, 16)])

        st_c[...] = sp_i
        pltpu.sync_copy(st_c, qsp.at[b, pl.ds(base, 16)])

        st_a[...] = b * HW + sp_i
        pltpu.sync_copy(xb.at[st_a], rows_xb)
        st_b[...] = labv
        pltpu.sync_copy(wceT.at[st_b], rows_wce)
        for p in range(16):
            for cc in range(8):
                rows_xb[p, pl.ds(cc * 16, 16)] = (
                    rows_xb[p, pl.ds(cc * 16, 16)]
                    + rows_wce[p, pl.ds(cc * 16, 16)]
                    + bce_l[pl.ds(cc * 16, 16)])
        pltpu.sync_copy(rows_xb, qf.at[b, pl.ds(base, 16)])


def _sc_select_gather(hm_sel, thr, xb, wceT, bce):
    i32 = jnp.int32
    f32 = jnp.float32
    mesh = plsc.VectorSubcoreMesh(core_axis_name="c", subcore_axis_name="s")
    fn = functools.partial(
        pl.kernel,
        mesh=mesh,
        out_type=[
            jax.ShapeDtypeStruct((B, NP_PAD, C_IN), f32),
            jax.ShapeDtypeStruct((B, 2, NP_PAD), f32),
            jax.ShapeDtypeStruct((B, NP_PAD), i32),
            jax.ShapeDtypeStruct((B, NP_PAD), i32),
        ],
        scratch_types=[
            pltpu.VMEM((CHUNK,), f32),
            pltpu.VMEM((CAND,), f32),
            pltpu.VMEM((CAND,), i32),
            pltpu.VMEM((NP_PAD,), i32),
            pltpu.VMEM((16,), i32),
            pltpu.VMEM((16,), i32),
            pltpu.VMEM((16,), i32),
            pltpu.VMEM((16,), f32),
            pltpu.VMEM((HITCAP * 16,), i32),
            pltpu.VMEM((16, C_IN), f32),
            pltpu.VMEM((16, C_IN), f32),
            pltpu.VMEM((C_IN,), f32),
            pltpu.VMEM_SHARED((CAND,), f32),
            pltpu.VMEM_SHARED((CAND,), i32),
            pltpu.VMEM_SHARED((NP_PAD,), i32),
            pltpu.SMEM((1,), i32),
        ],
    )
    return fn(_sc_body)(hm_sel, thr, xb, wceT, bce)


SG_CH = 4096

def _score_body(hm_ref, sp_ref, out_ref):
    f32 = jnp.float32
    i = pl.program_id(1)

    @pl.when(i == 0)
    def _():
        out_ref[0] = jnp.zeros((NP_PAD, NUM_CLASSES), f32)

    spv = sp_ref[0, 0]
    j = i * SG_CH + lax.broadcasted_iota(jnp.int32, (1, SG_CH), 1)
    oh = (spv[:, None] == j).astype(f32)
    out_ref[0] += lax.dot_general(
        oh, hm_ref[0], (((1,), (0,)), ((), ())),
        preferred_element_type=f32, precision=lax.Precision.HIGHEST)


def _score_gather(hm_rows3, qsp3):
    return pl.pallas_call(
        _score_body,
        grid=(B, HW // SG_CH),
        in_specs=[
            pl.BlockSpec((1, SG_CH, NUM_CLASSES), lambda b, i: (b, i, 0)),
            pl.BlockSpec((1, 1, NP_PAD), lambda b, i: (b, 0, 0)),
        ],
        out_specs=pl.BlockSpec((1, NP_PAD, NUM_CLASSES), lambda b, i: (b, 0, 0)),
        out_shape=jax.ShapeDtypeStruct((B, NP_PAD, NUM_CLASSES), jnp.float32),
    )(hm_rows3, qsp3)


def kernel(bev_feats, w1, bn_gamma, bn_beta, w2, b2, w_ce, b_ce, bev_pos):
    f32 = jnp.float32

    x_nhwc = bev_feats.transpose(0, 2, 3, 1)
    w1r = w1.transpose(2, 3, 1, 0).reshape(9 * C_IN, C_IN)
    w2r = w2.transpose(2, 3, 1, 0).reshape(9 * C_IN, NUM_CLASSES)
    scale = bn_gamma / jnp.sqrt(1.0 + 1e-5)

    dense_nhwc, hm_nhwc = _conv_nms(x_nhwc, w1r, w2r, scale, bn_beta, b2)
    dense_heatmap = dense_nhwc.transpose(0, 3, 1, 2)

    hm_sel = hm_nhwc.transpose(0, 3, 1, 2).reshape(B, N_SEL)
    xb = x_nhwc.reshape(B * HW, C_IN)
    wceT = w_ce.transpose(1, 0)

    thr = _thresholds(hm_sel.reshape(B, 640, 1024)).reshape(B, 128)
    qf, qpos, qlab, qsp = _sc_select_gather(hm_sel, thr, xb, wceT, b_ce)
    qhs = _score_gather(hm_nhwc.reshape(B, HW, NUM_CLASSES),
                        qsp.reshape(B, 1, NP_PAD))

    query_feat = qf[:, :NUM_PROPOSALS, :].transpose(0, 2, 1)
    query_pos = qpos.transpose(0, 2, 1)[:, :NUM_PROPOSALS, :]
    query_labels = qlab[:, :NUM_PROPOSALS]
    query_heatmap_score = qhs[:, :NUM_PROPOSALS, :].transpose(0, 2, 1)
    return (query_feat, query_pos, query_labels, query_heatmap_score, dense_heatmap)


# speedup vs baseline: 2.4598x; 1.2489x over previous
import functools

import jax
import jax.numpy as jnp
from jax import lax
from jax.experimental import pallas as pl
from jax.experimental.pallas import tpu as pltpu
from jax.experimental.pallas import tpu_sc as plsc

B = 2
C_IN = 128
H = 256
W = 256
NUM_CLASSES = 10
NUM_PROPOSALS = 500
HW = H * W
N_SEL = NUM_CLASSES * HW

R = 16
NB = H // R

NS = 16
CHUNK = N_SEL // NS
NBINS = 2048
CAP = 704
CAND = 768
NP_PAD = 512
HITCAP = 1024


def _conv_nms_body(x_prev, x_cur, x_next, w1r, w2r, scale, beta, b2,
                   dense_out, hm_out, halo, y1):
    i = pl.program_id(1)
    f32 = jnp.float32
    bf16 = jnp.bfloat16

    zcol = jnp.zeros((R + 6, 1, C_IN), f32)
    halo[:, 0:1, :] = zcol
    halo[:, 257:258, :] = zcol
    halo[3:3 + R, 1:257, :] = x_cur[0]
    halo[0:3, 1:257, :] = jnp.where(i > 0, x_prev[0, R - 3:R, :, :], 0.0)
    halo[3 + R:R + 6, 1:257, :] = jnp.where(i < NB - 1, x_next[0, 0:3, :, :], 0.0)

    a9 = jnp.concatenate(
        [halo[t // 3:t // 3 + R + 4, t % 3:t % 3 + 256, :] for t in range(9)],
        axis=2).astype(bf16)
    acc = lax.dot_general(a9, w1r[:].astype(bf16), (((2,), (0,)), ((), ())),
                          preferred_element_type=f32)
    y1c = jnp.maximum(acc * scale[:][None, None, :] + beta[:][None, None, :], 0.0)
    gr = i * R - 2 + lax.broadcasted_iota(jnp.int32, (R + 4, 1, 1), 0)
    y1c = jnp.where((gr >= 0) & (gr < H), y1c, 0.0)
    zcol1 = jnp.zeros((R + 4, 1, C_IN), f32)
    y1[:, 0:1, :] = zcol1
    y1[:, 257:258, :] = zcol1
    y1[:, 1:257, :] = y1c

    a9b = jnp.concatenate(
        [y1[t // 3:t // 3 + R + 2, t % 3:t % 3 + 256, :] for t in range(9)],
        axis=2).astype(bf16)
    acc2 = lax.dot_general(a9b, w2r[:].astype(bf16), (((2,), (0,)), ((), ())),
                           preferred_element_type=f32)
    y2 = acc2 + b2[:][None, None, :]
    dense_out[0] = y2[1:R + 1]

    s = jax.nn.sigmoid(y2)
    sp = jnp.pad(s, ((0, 0), (1, 1), (0, 0)))
    m = None
    for t in range(9):
        v = sp[t // 3:t // 3 + R, t % 3:t % 3 + 256, :]
        m = v if m is None else jnp.maximum(m, v)
    hmc = s[1:R + 1]
    grow = i * R + lax.broadcasted_iota(jnp.int32, (R, 1, 1), 0)
    interior_r = (grow >= 1) & (grow <= H - 2)
    col = lax.broadcasted_iota(jnp.int32, (1, 256, 1), 1)
    interior_c = (col >= 1) & (col <= W - 2)
    cls = lax.broadcasted_iota(jnp.int32, (1, 1, NUM_CLASSES), 2)
    keep = interior_r & interior_c & (hmc >= m)
    masked = jnp.where(keep, hmc, 0.0)
    hm_out[0] = jnp.where(cls >= 8, hmc, masked)


def _conv_nms(x_nhwc, w1r, w2r, scale, beta, b2):
    f32 = jnp.float32
    blk = lambda b, i: (b, i, 0, 0)
    return pl.pallas_call(
        _conv_nms_body,
        grid=(B, NB),
        in_specs=[
            pl.BlockSpec((1, R, W, C_IN), lambda b, i: (b, jnp.maximum(i - 1, 0), 0, 0)),
            pl.BlockSpec((1, R, W, C_IN), blk),
            pl.BlockSpec((1, R, W, C_IN), lambda b, i: (b, jnp.minimum(i + 1, NB - 1), 0, 0)),
            pl.BlockSpec((9 * C_IN, C_IN), lambda b, i: (0, 0)),
            pl.BlockSpec((9 * C_IN, NUM_CLASSES), lambda b, i: (0, 0)),
            pl.BlockSpec((C_IN,), lambda b, i: (0,)),
            pl.BlockSpec((C_IN,), lambda b, i: (0,)),
            pl.BlockSpec((NUM_CLASSES,), lambda b, i: (0,)),
        ],
        out_specs=[
            pl.BlockSpec((1, R, W, NUM_CLASSES), blk),
            pl.BlockSpec((1, R, W, NUM_CLASSES), blk),
        ],
        out_shape=[
            jax.ShapeDtypeStruct((B, H, W, NUM_CLASSES), f32),
            jax.ShapeDtypeStruct((B, H, W, NUM_CLASSES), f32),
        ],
        scratch_shapes=[
            pltpu.VMEM((R + 6, 258, C_IN), f32),
            pltpu.VMEM((R + 4, 258, C_IN), f32),
        ],
    )(x_nhwc, x_nhwc, x_nhwc, w1r, w2r, scale, beta, b2)


def _thresh_body(x_ref, out_ref):
    f32 = jnp.float32
    x = x_ref[0]

    def body(k, c):
        lo, hi = c
        t = (lo + hi) * 0.5
        cnt = jnp.sum((x >= t).astype(f32))
        big = cnt >= float(NUM_PROPOSALS)
        return (jnp.where(big, t, lo), jnp.where(big, hi, t))

    lo, _ = lax.fori_loop(0, 35, body, (jnp.float32(0.0), jnp.float32(1.0)))
    out_ref[0, 0] = jnp.full((128,), lo, f32)


def _thresholds(hm_sel3):
    return pl.pallas_call(
        _thresh_body,
        grid=(B,),
        in_specs=[pl.BlockSpec((1, 640, 1024), lambda b: (b, 0, 0))],
        out_specs=pl.BlockSpec((1, 1, 128), lambda b: (b, 0, 0)),
        out_shape=jax.ShapeDtypeStruct((B, 1, 128), jnp.float32),
    )(hm_sel3)


def _sc_body(hm_sel, thr, xb, wceT, bce,
             qf, qpos, qlab, qsp,
             v_chunk,
             cand_v, cand_i, sorted_l,
             st_a, st_b, st_c, st_d, hitlist,
             rows_xb, rows_wce, bce_l,
             sp_cval, sp_cidx, sp_sorted,
             cnt_smem):
    i32 = jnp.int32
    f32 = jnp.float32
    b = lax.axis_index("c")
    s = lax.axis_index("s")
    lane = lax.iota(i32, 16)

    pltpu.sync_copy(bce, bce_l)
    pltpu.sync_copy(hm_sel.at[b, pl.ds(s * CHUNK, CHUNK)], v_chunk)

    pltpu.sync_copy(thr.at[b, pl.ds(0, 16)], st_d)
    thr_v = st_d[...]

    @pl.when(s == 0)
    def _():
        st_b[...] = jnp.full((16,), 1 << 28, i32)
        st_c[...] = jnp.zeros((16,), i32)
        st_d[...] = jnp.full((16,), -1.0, f32)
        for k in range(CAND // 16):
            pltpu.sync_copy(st_b, sp_cidx.at[pl.ds(k * 16, 16)])
            pltpu.sync_copy(st_d, sp_cval.at[pl.ds(k * 16, 16)])
        for k in range(NP_PAD // 16):
            pltpu.sync_copy(st_c, sp_sorted.at[pl.ds(k * 16, 16)])
        cnt_smem[0] = 0
    plsc.subcore_barrier()

    def _vsum(vec_i32):
        t = vec_i32[0]
        for l in range(1, 16):
            t = t + vec_i32[l]
        return t

    def c_body(j, carry):
        acc, nhit = carry
        v = v_chunk[pl.ds(j * 16, 16)]
        m01 = jnp.where(v >= thr_v, 1, 0)
        c16 = _vsum(m01)

        @pl.when(c16 > 0)
        def _():
            hitlist[pl.ds(jnp.minimum(nhit, HITCAP - 1) * 16, 16)] = (
                jnp.full((16,), j, i32))
        nhit = jnp.minimum(nhit + jnp.where(c16 > 0, 1, 0), HITCAP - 1)
        return (acc + m01, nhit)
    cnt_vec, nhit = lax.fori_loop(0, CHUNK // 16, c_body,
                                  (jnp.zeros((16,), i32), 0))
    cnt = _vsum(cnt_vec)
    my_off = plsc.fetch_and_add(cnt_smem.at[0], cnt, subcore_id=0)

    def k_body(h, cur):
        j = hitlist[pl.ds(h * 16, 16)][0]
        v = v_chunk[pl.ds(j * 16, 16)]
        mask = v >= thr_v
        m01 = jnp.where(mask, 1, 0)
        gidx = s * CHUNK + j * 16 + lane
        pf = jnp.zeros((16,), i32)
        run = jnp.int32(0)
        for l in range(16):
            pf = jnp.where(lane == l, run, pf)
            run = run + m01[l]
        pos = my_off + cur + pf
        tgt = jnp.where(mask & (pos < CAP), pos, CAP + (pos & 63))
        st_a[...] = tgt
        st_d[...] = v
        pltpu.sync_copy(st_d, sp_cval.at[st_a])
        st_b[...] = gidx
        pltpu.sync_copy(st_b, sp_cidx.at[st_a])
        return cur + run
    lax.fori_loop(0, nhit, k_body, 0)
    plsc.subcore_barrier()

    pltpu.sync_copy(sp_cval, cand_v)
    pltpu.sync_copy(sp_cidx, cand_i)
    p0 = s * (CAND // NS)

    def pv_body(pv, _):
        vch = cand_v[pl.ds(p0 + pv * 16, 16)]
        ich = cand_i[pl.ds(p0 + pv * 16, 16)]
        rank_vec = jnp.zeros((16,), i32)
        for pj in range(16):
            vp = vch[pj]
            ip = ich[pj]

            def q_body(q, acc, vp=vp, ip=ip):
                vq = cand_v[pl.ds(q * 16, 16)]
                iq = cand_i[pl.ds(q * 16, 16)]
                pri = (vq > vp) | ((vq == vp) & (iq < ip))
                return acc + jnp.where(pri, 1, 0)
            acc = lax.fori_loop(0, CAND // 16, q_body, jnp.zeros((16,), i32))
            rank = _vsum(acc)
            rank_vec = jnp.where(lane == pj, rank, rank_vec)

        tgt = jnp.where(rank_vec < NUM_PROPOSALS, rank_vec,
                        NUM_PROPOSALS + (lane & 7))
        st_a[...] = tgt
        st_b[...] = ich
        pltpu.sync_copy(st_b, sp_sorted.at[st_a])
        return 0
    lax.fori_loop(0, CAND // NS // 16, pv_body, 0)
    plsc.subcore_barrier()

    pltpu.sync_copy(sp_sorted, sorted_l)
    for ch in range(2):
        base = (s * 2 + ch) * 16
        prop = sorted_l[pl.ds(base, 16)]
        sp_i = prop & (HW - 1)
        labv = lax.shift_right_logical(prop, 16)

        st_b[...] = labv
        pltpu.sync_copy(st_b, qlab.at[b, pl.ds(base, 16)])

        xv = (lax.shift_right_logical(sp_i, 8)).astype(f32) + 0.5
        yv = (sp_i & (W - 1)).astype(f32) + 0.5
        st_d[...] = xv
        pltpu.sync_copy(st_d, qpos.at[b, 0, pl.ds(base, 16)])
        st_d[...] = yv
        pltpu.sync_copy(st_d, qpos.at[b, 1, pl.ds(base, 16)])

        st_c[...] = sp_i
        pltpu.sync_copy(st_c, qsp.at[b, pl.ds(base, 16)])

        st_a[...] = b * HW + sp_i
        pltpu.sync_copy(xb.at[st_a], rows_xb)
        st_b[...] = labv
        pltpu.sync_copy(wceT.at[st_b], rows_wce)
        for p in range(16):
            for cc in range(8):
                rows_xb[p, pl.ds(cc * 16, 16)] = (
                    rows_xb[p, pl.ds(cc * 16, 16)]
                    + rows_wce[p, pl.ds(cc * 16, 16)]
                    + bce_l[pl.ds(cc * 16, 16)])
        pltpu.sync_copy(rows_xb, qf.at[b, pl.ds(base, 16)])


def _sc_select_gather(hm_sel, thr, xb, wceT, bce):
    i32 = jnp.int32
    f32 = jnp.float32
    mesh = plsc.VectorSubcoreMesh(core_axis_name="c", subcore_axis_name="s")
    fn = functools.partial(
        pl.kernel,
        mesh=mesh,
        out_type=[
            jax.ShapeDtypeStruct((B, NP_PAD, C_IN), f32),
            jax.ShapeDtypeStruct((B, 2, NP_PAD), f32),
            jax.ShapeDtypeStruct((B, NP_PAD), i32),
            jax.ShapeDtypeStruct((B, NP_PAD), i32),
        ],
        scratch_types=[
            pltpu.VMEM((CHUNK,), f32),
            pltpu.VMEM((CAND,), f32),
            pltpu.VMEM((CAND,), i32),
            pltpu.VMEM((NP_PAD,), i32),
            pltpu.VMEM((16,), i32),
            pltpu.VMEM((16,), i32),
            pltpu.VMEM((16,), i32),
            pltpu.VMEM((16,), f32),
            pltpu.VMEM((HITCAP * 16,), i32),
            pltpu.VMEM((16, C_IN), f32),
            pltpu.VMEM((16, C_IN), f32),
            pltpu.VMEM((C_IN,), f32),
            pltpu.VMEM_SHARED((CAND,), f32),
            pltpu.VMEM_SHARED((CAND,), i32),
            pltpu.VMEM_SHARED((NP_PAD,), i32),
            pltpu.SMEM((1,), i32),
        ],
    )
    return fn(_sc_body)(hm_sel, thr, xb, wceT, bce)


SG_CH = 4096

def _score_body(hm_ref, sp_ref, out_ref):
    f32 = jnp.float32
    i = pl.program_id(1)

    @pl.when(i == 0)
    def _():
        out_ref[0] = jnp.zeros((NUM_CLASSES, NP_PAD), f32)

    spv = sp_ref[0, 0]
    j = i * SG_CH + lax.broadcasted_iota(jnp.int32, (SG_CH, 1), 0)
    oh = (j == spv[None, :]).astype(f32)
    out_ref[0] += lax.dot_general(
        hm_ref[0], oh, (((0,), (0,)), ((), ())),
        preferred_element_type=f32, precision=lax.Precision.HIGHEST)


def _score_gather(hm_rows3, qsp3):
    return pl.pallas_call(
        _score_body,
        grid=(B, HW // SG_CH),
        in_specs=[
            pl.BlockSpec((1, SG_CH, NUM_CLASSES), lambda b, i: (b, i, 0)),
            pl.BlockSpec((1, 1, NP_PAD), lambda b, i: (b, 0, 0)),
        ],
        out_specs=pl.BlockSpec((1, NUM_CLASSES, NP_PAD), lambda b, i: (b, 0, 0)),
        out_shape=jax.ShapeDtypeStruct((B, NUM_CLASSES, NP_PAD), jnp.float32),
    )(hm_rows3, qsp3)


def kernel(bev_feats, w1, bn_gamma, bn_beta, w2, b2, w_ce, b_ce, bev_pos):
    f32 = jnp.float32

    x_nhwc = bev_feats.transpose(0, 2, 3, 1)
    w1r = w1.transpose(2, 3, 1, 0).reshape(9 * C_IN, C_IN)
    w2r = w2.transpose(2, 3, 1, 0).reshape(9 * C_IN, NUM_CLASSES)
    scale = bn_gamma / jnp.sqrt(1.0 + 1e-5)

    dense_nhwc, hm_nhwc = _conv_nms(x_nhwc, w1r, w2r, scale, bn_beta, b2)
    dense_heatmap = dense_nhwc.transpose(0, 3, 1, 2)

    hm_sel = hm_nhwc.transpose(0, 3, 1, 2).reshape(B, N_SEL)
    xb = x_nhwc.reshape(B * HW, C_IN)
    wceT = w_ce.transpose(1, 0)

    thr = _thresholds(hm_sel.reshape(B, 640, 1024)).reshape(B, 128)
    qf, qpos, qlab, qsp = _sc_select_gather(hm_sel, thr, xb, wceT, b_ce)
    qhs = _score_gather(hm_nhwc.reshape(B, HW, NUM_CLASSES),
                        qsp.reshape(B, 1, NP_PAD))

    query_feat = qf[:, :NUM_PROPOSALS, :].transpose(0, 2, 1)
    query_pos = qpos.transpose(0, 2, 1)[:, :NUM_PROPOSALS, :]
    query_labels = qlab[:, :NUM_PROPOSALS]
    query_heatmap_score = qhs[:, :, :NUM_PROPOSALS]
    return (query_feat, query_pos, query_labels, query_heatmap_score, dense_heatmap)
